# Initial kernel scaffold; baseline (speedup 1.0000x reference)
import jax
import jax.numpy as jnp
from jax.experimental import pallas as pl


def kernel(x, W, b):
    raise NotImplementedError("write your pallas kernel here")


# speedup vs baseline: 6.2558x; 6.2558x over previous
import jax
import jax.numpy as jnp
from jax.experimental import pallas as pl

DIM = 768
NUM_EXPERTS = 64
CAPACITY_FACTOR = 1.0
EPSILON = 1e-06

BM = 2048


def _stage_a(x_ref, w_ref, b_ref, score_ref, amax_ref, denom_ref):
    j = pl.program_id(0)
    logits = jax.lax.dot_general(
        x_ref[...], w_ref[...],
        (((1,), (1,)), ((), ())),
        preferred_element_type=jnp.float32,
    ) + b_ref[...]
    m = jnp.max(logits, axis=1, keepdims=True)
    idx = jax.lax.broadcasted_iota(jnp.int32, logits.shape, 1)
    a = jnp.min(jnp.where(logits == m, idx, NUM_EXPERTS), axis=1)
    s = jnp.sum(jnp.exp(logits - m), axis=1)
    score = 1.0 / s
    score_ref[0, 0, :] = score
    amax_ref[0, 0, :] = a
    onehot = (idx == a[:, None]).astype(jnp.float32)
    contrib = jnp.sum(onehot * score[:, None], axis=0)[None, :]

    @pl.when(j == 0)
    def _():
        denom_ref[...] = jnp.zeros_like(denom_ref)

    denom_ref[...] += contrib


def _stage_b(score_ref, amax_ref, denom_ref, out_ref):
    score = score_ref[0, 0, :]
    a = amax_ref[0, 0, :]
    capacity = jnp.float32(out_ref.shape[0] * pl.num_programs(0) * CAPACITY_FACTOR)
    inv = capacity / (denom_ref[0, :] + EPSILON)
    idx = jax.lax.broadcasted_iota(jnp.int32, (out_ref.shape[0], NUM_EXPERTS), 1)
    onehot = idx == a[:, None]
    out_ref[...] = jnp.where(onehot, score[:, None] * inv[None, :], 0.0)


def kernel(x, W, b):
    batch, N, dim = x.shape
    rows = batch * N
    xf = x.reshape(rows, dim)
    nb = rows // BM
    b2 = b.reshape(1, NUM_EXPERTS)

    score, amax, denom = pl.pallas_call(
        _stage_a,
        grid=(nb,),
        in_specs=[
            pl.BlockSpec((BM, dim), lambda j: (j, 0)),
            pl.BlockSpec((NUM_EXPERTS, dim), lambda j: (0, 0)),
            pl.BlockSpec((1, NUM_EXPERTS), lambda j: (0, 0)),
        ],
        out_specs=[
            pl.BlockSpec((1, 1, BM), lambda j: (j, 0, 0)),
            pl.BlockSpec((1, 1, BM), lambda j: (j, 0, 0)),
            pl.BlockSpec((1, NUM_EXPERTS), lambda j: (0, 0)),
        ],
        out_shape=[
            jax.ShapeDtypeStruct((nb, 1, BM), jnp.float32),
            jax.ShapeDtypeStruct((nb, 1, BM), jnp.int32),
            jax.ShapeDtypeStruct((1, NUM_EXPERTS), jnp.float32),
        ],
    )(xf, W, b2)

    out = pl.pallas_call(
        _stage_b,
        grid=(nb,),
        in_specs=[
            pl.BlockSpec((1, 1, BM), lambda j: (j, 0, 0)),
            pl.BlockSpec((1, 1, BM), lambda j: (j, 0, 0)),
            pl.BlockSpec((1, NUM_EXPERTS), lambda j: (0, 0)),
        ],
        out_specs=pl.BlockSpec((BM, NUM_EXPERTS), lambda j: (j, 0)),
        out_shape=jax.ShapeDtypeStruct((rows, NUM_EXPERTS), jnp.float32),
    )(score, amax, denom)

    return out.reshape(batch, N, NUM_EXPERTS)


# speedup vs baseline: 8.7155x; 1.3932x over previous
import jax
import jax.numpy as jnp
from jax.experimental import pallas as pl

DIM = 768
NUM_EXPERTS = 64
CAPACITY_FACTOR = 1.0
EPSILON = 1e-06

BM = 2048


def _stage_a(x_ref, w_ref, b_ref, score_ref, amax_ref, denom_ref):
    j = pl.program_id(0)
    xb = x_ref[0]
    logits = jax.lax.dot_general(
        w_ref[...], xb,
        (((1,), (1,)), ((), ())),
        preferred_element_type=jnp.float32,
    ) + b_ref[...]
    m = jnp.max(logits, axis=0, keepdims=True)
    idx = jax.lax.broadcasted_iota(jnp.int32, logits.shape, 0)
    a = jnp.min(jnp.where(logits == m, idx, NUM_EXPERTS), axis=0)
    s = jnp.sum(jnp.exp(logits - m), axis=0)
    score = 1.0 / s
    score_ref[0, 0, :] = score
    amax_ref[0, 0, :] = a
    onehot = (idx == a[None, :]).astype(jnp.float32)
    contrib = jax.lax.dot_general(
        onehot * score[None, :], jnp.ones((BM, 1), jnp.float32),
        (((1,), (0,)), ((), ())),
        preferred_element_type=jnp.float32,
    )

    @pl.when(j == 0)
    def _():
        denom_ref[...] = jnp.zeros_like(denom_ref)

    denom_ref[...] += contrib


def _stage_b(score_ref, amax_ref, denom_ref, out_ref):
    score = score_ref[0, 0, :]
    a = amax_ref[0, 0, :]
    capacity = jnp.float32(BM * pl.num_programs(0) * CAPACITY_FACTOR)
    inv = capacity / (denom_ref[:, 0] + EPSILON)
    idx = jax.lax.broadcasted_iota(jnp.int32, (BM, NUM_EXPERTS), 1)
    onehot = idx == a[:, None]
    out_ref[0] = jnp.where(onehot, score[:, None] * inv[None, :], 0.0)


def kernel(x, W, b):
    batch, N, dim = x.shape
    rows = batch * N
    nb = rows // BM
    per_batch = N // BM
    b2 = b.reshape(NUM_EXPERTS, 1)

    score, amax, denom = pl.pallas_call(
        _stage_a,
        grid=(nb,),
        in_specs=[
            pl.BlockSpec((1, BM, dim), lambda j: (j // per_batch, j % per_batch, 0)),
            pl.BlockSpec((NUM_EXPERTS, dim), lambda j: (0, 0)),
            pl.BlockSpec((NUM_EXPERTS, 1), lambda j: (0, 0)),
        ],
        out_specs=[
            pl.BlockSpec((1, 1, BM), lambda j: (j, 0, 0)),
            pl.BlockSpec((1, 1, BM), lambda j: (j, 0, 0)),
            pl.BlockSpec((NUM_EXPERTS, 1), lambda j: (0, 0)),
        ],
        out_shape=[
            jax.ShapeDtypeStruct((nb, 1, BM), jnp.float32),
            jax.ShapeDtypeStruct((nb, 1, BM), jnp.int32),
            jax.ShapeDtypeStruct((NUM_EXPERTS, 1), jnp.float32),
        ],
    )(x, W, b2)

    out = pl.pallas_call(
        _stage_b,
        grid=(nb,),
        in_specs=[
            pl.BlockSpec((1, 1, BM), lambda j: (j, 0, 0)),
            pl.BlockSpec((1, 1, BM), lambda j: (j, 0, 0)),
            pl.BlockSpec((NUM_EXPERTS, 1), lambda j: (0, 0)),
        ],
        out_specs=pl.BlockSpec(
            (1, BM, NUM_EXPERTS), lambda j: (j // per_batch, j % per_batch, 0)
        ),
        out_shape=jax.ShapeDtypeStruct((batch, N, NUM_EXPERTS), jnp.float32),
    )(score, amax, denom)

    return out


# speedup vs baseline: 9.4699x; 1.0866x over previous
import jax
import jax.numpy as jnp
from jax.experimental import pallas as pl

DIM = 768
NUM_EXPERTS = 64
CAPACITY_FACTOR = 1.0
EPSILON = 1e-06

BM = 4096


def _stage_a(x_ref, w_ref, b_ref, score_ref, amax_ref, denom_ref):
    j = pl.program_id(0)
    xb = x_ref[0]
    logits = jax.lax.dot_general(
        w_ref[...], xb,
        (((1,), (1,)), ((), ())),
        preferred_element_type=jnp.float32,
    ) + b_ref[...]
    m = jnp.max(logits, axis=0, keepdims=True)
    idx = jax.lax.broadcasted_iota(jnp.int32, logits.shape, 0)
    a = jnp.min(jnp.where(logits == m, idx, NUM_EXPERTS), axis=0)
    s = jnp.sum(jnp.exp(logits - m), axis=0)
    score = 1.0 / s
    score_ref[0, 0, :] = score
    amax_ref[0, 0, :] = a
    onehot = (idx == a[None, :]).astype(jnp.float32)
    contrib = jax.lax.dot_general(
        onehot * score[None, :], jnp.ones((BM, 1), jnp.float32),
        (((1,), (0,)), ((), ())),
        preferred_element_type=jnp.float32,
    )

    @pl.when(j == 0)
    def _():
        denom_ref[...] = jnp.zeros_like(denom_ref)

    denom_ref[...] += contrib


def _stage_b(score_ref, amax_ref, denom_ref, out_ref):
    score = score_ref[0, 0, :]
    a = amax_ref[0, 0, :]
    capacity = jnp.float32(BM * pl.num_programs(0) * CAPACITY_FACTOR)
    inv = capacity / (denom_ref[:, 0] + EPSILON)
    idx = jax.lax.broadcasted_iota(jnp.int32, (BM, NUM_EXPERTS), 1)
    onehot = idx == a[:, None]
    out_ref[0] = jnp.where(onehot, score[:, None] * inv[None, :], 0.0)


def kernel(x, W, b):
    batch, N, dim = x.shape
    rows = batch * N
    nb = rows // BM
    per_batch = N // BM
    b2 = b.reshape(NUM_EXPERTS, 1)

    score, amax, denom = pl.pallas_call(
        _stage_a,
        grid=(nb,),
        in_specs=[
            pl.BlockSpec((1, BM, dim), lambda j: (j // per_batch, j % per_batch, 0)),
            pl.BlockSpec((NUM_EXPERTS, dim), lambda j: (0, 0)),
            pl.BlockSpec((NUM_EXPERTS, 1), lambda j: (0, 0)),
        ],
        out_specs=[
            pl.BlockSpec((1, 1, BM), lambda j: (j, 0, 0)),
            pl.BlockSpec((1, 1, BM), lambda j: (j, 0, 0)),
            pl.BlockSpec((NUM_EXPERTS, 1), lambda j: (0, 0)),
        ],
        out_shape=[
            jax.ShapeDtypeStruct((nb, 1, BM), jnp.float32),
            jax.ShapeDtypeStruct((nb, 1, BM), jnp.int32),
            jax.ShapeDtypeStruct((NUM_EXPERTS, 1), jnp.float32),
        ],
    )(x, W, b2)

    out = pl.pallas_call(
        _stage_b,
        grid=(nb,),
        in_specs=[
            pl.BlockSpec((1, 1, BM), lambda j: (j, 0, 0)),
            pl.BlockSpec((1, 1, BM), lambda j: (j, 0, 0)),
            pl.BlockSpec((NUM_EXPERTS, 1), lambda j: (0, 0)),
        ],
        out_specs=pl.BlockSpec(
            (1, BM, NUM_EXPERTS), lambda j: (j // per_batch, j % per_batch, 0)
        ),
        out_shape=jax.ShapeDtypeStruct((batch, N, NUM_EXPERTS), jnp.float32),
    )(score, amax, denom)

    return out
